# baseline (speedup 1.0000x reference)
import functools

import jax
import jax.numpy as jnp
from jax import lax
from jax.experimental import pallas as pl
from jax.experimental.pallas import tpu as pltpu
from jax.experimental.pallas import tpu_sc as plsc

CHUNK = 128
NBUF = 4


@functools.lru_cache(maxsize=None)
def _make_kernel(B, D, nw, b_per_w, n_chunks):
    mesh = plsc.VectorSubcoreMesh(core_axis_name="c", subcore_axis_name="s")
    nc = mesh.num_cores
    n_groups = n_chunks // NBUF

    @functools.partial(
        pl.kernel,
        mesh=mesh,
        compiler_params=pltpu.CompilerParams(use_tc_tiling_on_sc=False),
        out_type=jax.ShapeDtypeStruct((B, D), jnp.float32),
        scratch_types=(
            [
                pltpu.VMEM((n_chunks, CHUNK), jnp.int32),
                pltpu.VMEM((NBUF, CHUNK, D), jnp.float32),
            ]
            + [pltpu.SemaphoreType.DMA] * (2 * NBUF)
        ),
    )
    def emb(idx_hbm, table_hbm, out_hbm, idx_v, rows_v, *sems):
        gsems = sems[:NBUF]
        wsems = sems[NBUF:]
        wid = lax.axis_index("s") * nc + lax.axis_index("c")
        base = wid * b_per_w

        pltpu.sync_copy(idx_hbm.at[wid], idx_v)

        def start_gather(j, b):
            pltpu.async_copy(table_hbm.at[idx_v.at[j]], rows_v.at[b], gsems[b])

        def wait_gather(j, b):
            pltpu.make_async_copy(
                table_hbm.at[idx_v.at[j]], rows_v.at[b], gsems[b]
            ).wait()

        def out_slice(j):
            return out_hbm.at[pl.ds(base + j * CHUNK, CHUNK)]

        def start_write(j, b):
            pltpu.async_copy(rows_v.at[b], out_slice(j), wsems[b])

        def wait_write(j, b):
            pltpu.make_async_copy(rows_v.at[b], out_slice(j), wsems[b]).wait()

        for b in range(NBUF):
            start_gather(b, b)

        def body(g, carry):
            for b in range(NBUF):
                j = g * NBUF + b
                wait_gather(j, b)
                start_write(j, b)
            for b in range(NBUF):
                j = g * NBUF + b
                wait_write(j, b)
                start_gather(j + NBUF, b)
            return carry

        lax.fori_loop(0, n_groups - 1, body, 0)

        g = n_groups - 1
        for b in range(NBUF):
            j = g * NBUF + b
            wait_gather(j, b)
            start_write(j, b)
        for b in range(NBUF):
            j = g * NBUF + b
            wait_write(j, b)

    return emb


def kernel(x, table):
    orig_shape = x.shape
    B = x.shape[0] * x.shape[1]
    D = table.shape[1]

    info = plsc.get_sparse_core_info()
    nw = info.num_cores * info.num_subcores
    b_per_w = B // nw
    n_chunks = b_per_w // CHUNK
    assert b_per_w * nw == B and n_chunks * CHUNK == b_per_w

    idx = x.reshape(nw, n_chunks, CHUNK).astype(jnp.int32)
    emb = _make_kernel(B, D, nw, b_per_w, n_chunks)
    out = emb(idx, table)
    return out.reshape(*orig_shape, D)
